# baseline (speedup 1.0000x reference)
import functools

import jax
import jax.numpy as jnp
from jax import lax
from jax.experimental import pallas as pl
from jax.experimental.pallas import tpu as pltpu
from jax.experimental.pallas import tpu_sc as plsc

NW = 32
CW = 80
NPAD = 10240

_MESH = plsc.VectorSubcoreMesh(core_axis_name="c", subcore_axis_name="s")


def _make_deg_kernel(NP, NCH):
    @functools.partial(
        pl.kernel,
        mesh=_MESH,
        out_type=jax.ShapeDtypeStruct((2, NP), jnp.float32),
        compiler_params=pltpu.CompilerParams(use_tc_tiling_on_sc=False),
        scratch_types=[
            pltpu.VMEM((NCH, CW), jnp.int32),
            pltpu.VMEM((CW,), jnp.float32),
            pltpu.VMEM_SHARED((NP,), jnp.float32),
            pltpu.SemaphoreType.DMA,
        ],
    )
    def deg_kernel(dst_hbm, zeros_hbm, out_hbm, idx_v, ones_v, acc, sem):
        cid = lax.axis_index("c")
        sid = lax.axis_index("s")
        wid = sid * 2 + cid
        rpw = NP // 16

        pltpu.sync_copy(zeros_hbm.at[pl.ds(sid * rpw, rpw)],
                        acc.at[pl.ds(sid * rpw, rpw)])

        for t in range(CW // 16):
            ones_v[pl.ds(t * 16, 16)] = jnp.ones((16,), jnp.float32)
        pltpu.sync_copy(dst_hbm.at[wid], idx_v)
        plsc.subcore_barrier()

        pltpu.async_copy(ones_v, acc.at[idx_v.at[0]], sem, add=True)

        def body(j, carry):
            pltpu.async_copy(ones_v, acc.at[idx_v.at[j]], sem, add=True)
            pltpu.make_async_copy(ones_v, acc.at[idx_v.at[j - 1]], sem).wait()
            return carry

        lax.fori_loop(1, NCH, body, 0)
        pltpu.make_async_copy(ones_v, acc.at[idx_v.at[NCH - 1]], sem).wait()
        plsc.subcore_barrier()

        pltpu.sync_copy(acc.at[pl.ds(sid * rpw, rpw)],
                        out_hbm.at[cid].at[pl.ds(sid * rpw, rpw)])

    return deg_kernel


def _make_agg_kernel(NP, NCH, D):
    @functools.partial(
        pl.kernel,
        mesh=_MESH,
        out_type=jax.ShapeDtypeStruct((2, NP, D), jnp.float32),
        compiler_params=pltpu.CompilerParams(use_tc_tiling_on_sc=False),
        scratch_types=[
            pltpu.VMEM((NCH, CW), jnp.int32),
            pltpu.VMEM((NCH, CW), jnp.int32),
            pltpu.VMEM((CW, D), jnp.float32),
            pltpu.VMEM((CW, D), jnp.float32),
            pltpu.VMEM_SHARED((NP, D), jnp.float32),
            pltpu.VMEM_SHARED((NP, D), jnp.float32),
            pltpu.SemaphoreType.DMA,
            pltpu.SemaphoreType.DMA,
        ],
    )
    def agg_kernel(tab_hbm, src_hbm, dst_hbm, zeros_hbm, out_hbm,
                   src_v, dst_v, rows_a, rows_b, acc, tab_spm, sem_a, sem_b):
        cid = lax.axis_index("c")
        sid = lax.axis_index("s")
        wid = sid * 2 + cid
        rpw = NP // 16

        pltpu.sync_copy(zeros_hbm.at[pl.ds(sid * rpw, rpw)],
                        acc.at[pl.ds(sid * rpw, rpw)])

        pltpu.sync_copy(tab_hbm.at[pl.ds(sid * rpw, rpw)],
                        tab_spm.at[pl.ds(sid * rpw, rpw)])
        pltpu.sync_copy(src_hbm.at[wid], src_v)
        pltpu.sync_copy(dst_hbm.at[wid], dst_v)
        plsc.subcore_barrier()

        pltpu.async_copy(tab_spm.at[src_v.at[0]], rows_a, sem_a)

        def body(g, carry):
            j0 = 2 * g
            j1 = j0 + 1
            pltpu.make_async_copy(tab_spm.at[src_v.at[j0]], rows_a, sem_a).wait()
            pltpu.async_copy(tab_spm.at[src_v.at[j1]], rows_b, sem_b)
            pltpu.sync_copy(rows_a, acc.at[dst_v.at[j0]], add=True)
            pltpu.make_async_copy(tab_spm.at[src_v.at[j1]], rows_b, sem_b).wait()
            pltpu.async_copy(tab_spm.at[src_v.at[j0 + 2]], rows_a, sem_a)
            pltpu.sync_copy(rows_b, acc.at[dst_v.at[j1]], add=True)
            return carry

        lax.fori_loop(0, (NCH - 1) // 2, body, 0)
        pltpu.make_async_copy(
            tab_spm.at[src_v.at[NCH - 1]], rows_a, sem_a).wait()
        pltpu.sync_copy(rows_a, acc.at[dst_v.at[NCH - 1]], add=True)
        plsc.subcore_barrier()

        pltpu.sync_copy(acc.at[pl.ds(sid * rpw, rpw)],
                        out_hbm.at[cid].at[pl.ds(sid * rpw, rpw)])

    return agg_kernel


def _mm1_body(x2_ref, w_ref, o_ref):
    o_ref[...] = jnp.dot(x2_ref[...], w_ref[...],
                         preferred_element_type=jnp.float32)


def _scale_body(degp4_ref, h_ref, m64_ref, o_ref):
    deg = degp4_ref[0] + degp4_ref[1] + 1.0
    d64 = jnp.dot(lax.rsqrt(deg), m64_ref[...],
                  preferred_element_type=jnp.float32,
                  precision=lax.Precision.HIGHEST)
    o_ref[...] = h_ref[...] * d64


def _mid_body(degp4_ref, s_ref, hs_ref, b_ref, w_ref, m64_ref, m32_ref,
              o_ref):
    deg = degp4_ref[0] + degp4_ref[1] + 1.0
    dinv = lax.rsqrt(deg)
    d64 = jnp.dot(dinv, m64_ref[...], preferred_element_type=jnp.float32,
                  precision=lax.Precision.HIGHEST)
    pre = (s_ref[0] + s_ref[1] + hs_ref[...]) * d64 + b_ref[...]
    hmid = jnp.where(pre >= 0.0, pre, 0.01 * pre)
    h2 = jnp.dot(hmid, w_ref[...], preferred_element_type=jnp.float32)
    d32 = jnp.dot(dinv, m32_ref[...], preferred_element_type=jnp.float32,
                  precision=lax.Precision.HIGHEST)
    o_ref[...] = h2 * d32


def _final_body(degp4_ref, s_ref, hs_ref, b_ref, m32_ref, o_ref):
    deg = degp4_ref[0] + degp4_ref[1] + 1.0
    d32 = jnp.dot(lax.rsqrt(deg), m32_ref[...],
                  preferred_element_type=jnp.float32,
                  precision=lax.Precision.HIGHEST)
    pre = (s_ref[0] + s_ref[1] + hs_ref[...]) * d32 + b_ref[...]
    o_ref[...] = jnp.where(pre >= 0.0, pre, 0.01 * pre)


def kernel(x, edge_index, W1, b1, W2, b2):
    N, d_in = x.shape
    E = edge_index.shape[1]
    h1 = W1.shape[1]
    h2 = W2.shape[1]
    NP = NPAD
    NCH = E // (NW * CW)

    src = edge_index[0].reshape(NW, NCH, CW)
    dst = edge_index[1].reshape(NW, NCH, CW)

    W1d2 = (jnp.zeros((2 * d_in, 2 * h1), jnp.float32)
            .at[:d_in, :h1].set(W1)
            .at[d_in:, h1:].set(W1))
    W2d4 = jnp.zeros((4 * h1, 4 * h2), jnp.float32)
    for j in range(4):
        W2d4 = W2d4.at[j * h1:(j + 1) * h1, j * h2:(j + 1) * h2].set(W2)
    m64 = jnp.repeat(jnp.eye(4, dtype=jnp.float32), h1, axis=1)
    m32 = jnp.repeat(jnp.eye(4, dtype=jnp.float32), h2, axis=1)
    b1q = jnp.tile(b1, 4).reshape(1, 4 * h1)
    b2q = jnp.tile(b2, 4).reshape(1, 4 * h2)

    degp = _make_deg_kernel(NP, NCH)(dst, jnp.zeros((NP,), jnp.float32))
    degp4 = degp.reshape(2, NP // 4, 4)

    BNp = 1000
    BNq = 640
    x2 = x.reshape(N // 2, 2 * d_in)
    h1p = pl.pallas_call(
        _mm1_body,
        grid=(N // 2 // BNp,),
        in_specs=[
            pl.BlockSpec((BNp, 2 * d_in), lambda i: (i, 0)),
            pl.BlockSpec((2 * d_in, 2 * h1), lambda i: (0, 0)),
        ],
        out_specs=pl.BlockSpec((BNp, 2 * h1), lambda i: (i, 0)),
        out_shape=jax.ShapeDtypeStruct((NP // 2, 2 * h1), jnp.float32),
    )(x2, W1d2)

    h1q = h1p.reshape(NP // 4, 4 * h1)
    hs1q = pl.pallas_call(
        _scale_body,
        grid=(NP // 4 // BNq,),
        in_specs=[
            pl.BlockSpec((2, BNq, 4), lambda i: (0, i, 0)),
            pl.BlockSpec((BNq, 4 * h1), lambda i: (i, 0)),
            pl.BlockSpec((4, 4 * h1), lambda i: (0, 0)),
        ],
        out_specs=pl.BlockSpec((BNq, 4 * h1), lambda i: (i, 0)),
        out_shape=jax.ShapeDtypeStruct((NP // 4, 4 * h1), jnp.float32),
    )(degp4, h1q, m64)

    tab1 = hs1q.reshape(NP, h1)
    s1p = _make_agg_kernel(NP, NCH, h1)(
        tab1, src, dst, jnp.zeros((NP, h1), jnp.float32))

    s1q = s1p.reshape(2, NP // 4, 4 * h1)
    hs2q = pl.pallas_call(
        _mid_body,
        grid=(NP // 4 // BNq,),
        in_specs=[
            pl.BlockSpec((2, BNq, 4), lambda i: (0, i, 0)),
            pl.BlockSpec((2, BNq, 4 * h1), lambda i: (0, i, 0)),
            pl.BlockSpec((BNq, 4 * h1), lambda i: (i, 0)),
            pl.BlockSpec((1, 4 * h1), lambda i: (0, 0)),
            pl.BlockSpec((4 * h1, 4 * h2), lambda i: (0, 0)),
            pl.BlockSpec((4, 4 * h1), lambda i: (0, 0)),
            pl.BlockSpec((4, 4 * h2), lambda i: (0, 0)),
        ],
        out_specs=pl.BlockSpec((BNq, 4 * h2), lambda i: (i, 0)),
        out_shape=jax.ShapeDtypeStruct((NP // 4, 4 * h2), jnp.float32),
    )(degp4, s1q, hs1q, b1q, W2d4, m64, m32)

    tab2 = hs2q.reshape(NP, h2)
    s2p = _make_agg_kernel(NP, NCH, h2)(
        tab2, src, dst, jnp.zeros((NP, h2), jnp.float32))

    s2q = s2p.reshape(2, NP // 4, 4 * h2)
    outq = pl.pallas_call(
        _final_body,
        grid=(NP // 4 // BNq,),
        in_specs=[
            pl.BlockSpec((2, BNq, 4), lambda i: (0, i, 0)),
            pl.BlockSpec((2, BNq, 4 * h2), lambda i: (0, i, 0)),
            pl.BlockSpec((BNq, 4 * h2), lambda i: (i, 0)),
            pl.BlockSpec((1, 4 * h2), lambda i: (0, 0)),
            pl.BlockSpec((4, 4 * h2), lambda i: (0, 0)),
        ],
        out_specs=pl.BlockSpec((BNq, 4 * h2), lambda i: (i, 0)),
        out_shape=jax.ShapeDtypeStruct((NP // 4, 4 * h2), jnp.float32),
    )(degp4, s2q, hs2q, b2q, m32)

    return outq[:N // 4].reshape(N, h2)


# speedup vs baseline: 1.0173x; 1.0173x over previous
import functools

import jax
import jax.numpy as jnp
from jax import lax
from jax.experimental import pallas as pl
from jax.experimental.pallas import tpu as pltpu
from jax.experimental.pallas import tpu_sc as plsc

NW = 32
CW = 80
NPAD = 10240

_MESH = plsc.VectorSubcoreMesh(core_axis_name="c", subcore_axis_name="s")


def _make_deg_kernel(NP, NCH):
    @functools.partial(
        pl.kernel,
        mesh=_MESH,
        out_type=jax.ShapeDtypeStruct((2, NP), jnp.float32),
        compiler_params=pltpu.CompilerParams(use_tc_tiling_on_sc=False),
        scratch_types=[
            pltpu.VMEM((NCH, CW), jnp.int32),
            pltpu.VMEM((CW,), jnp.float32),
            pltpu.VMEM_SHARED((NP,), jnp.float32),
            pltpu.SemaphoreType.DMA,
        ],
    )
    def deg_kernel(dst_hbm, zeros_hbm, out_hbm, idx_v, ones_v, acc, sem):
        cid = lax.axis_index("c")
        sid = lax.axis_index("s")
        wid = sid * 2 + cid
        rpw = NP // 16

        pltpu.sync_copy(zeros_hbm.at[pl.ds(sid * rpw, rpw)],
                        acc.at[pl.ds(sid * rpw, rpw)])

        for t in range(CW // 16):
            ones_v[pl.ds(t * 16, 16)] = jnp.ones((16,), jnp.float32)
        pltpu.sync_copy(dst_hbm.at[wid], idx_v)
        plsc.subcore_barrier()

        pltpu.async_copy(ones_v, acc.at[idx_v.at[0]], sem, add=True)

        def body(j, carry):
            pltpu.async_copy(ones_v, acc.at[idx_v.at[j]], sem, add=True)
            pltpu.make_async_copy(ones_v, acc.at[idx_v.at[j - 1]], sem).wait()
            return carry

        lax.fori_loop(1, NCH, body, 0)
        pltpu.make_async_copy(ones_v, acc.at[idx_v.at[NCH - 1]], sem).wait()
        plsc.subcore_barrier()

        pltpu.sync_copy(acc.at[pl.ds(sid * rpw, rpw)],
                        out_hbm.at[cid].at[pl.ds(sid * rpw, rpw)])

    return deg_kernel


def _make_agg_kernel(NP, NCH, D):
    @functools.partial(
        pl.kernel,
        mesh=_MESH,
        out_type=jax.ShapeDtypeStruct((2, NP, D), jnp.float32),
        compiler_params=pltpu.CompilerParams(use_tc_tiling_on_sc=False),
        scratch_types=[
            pltpu.VMEM((NCH, CW), jnp.int32),
            pltpu.VMEM((NCH, CW), jnp.int32),
            pltpu.VMEM((CW, D), jnp.float32),
            pltpu.VMEM((CW, D), jnp.float32),
            pltpu.VMEM_SHARED((NP, D), jnp.float32),
            pltpu.VMEM_SHARED((NP, D), jnp.float32),
            pltpu.SemaphoreType.DMA,
            pltpu.SemaphoreType.DMA,
        ],
    )
    def agg_kernel(tab_hbm, src_hbm, dst_hbm, zeros_hbm, out_hbm,
                   src_v, dst_v, rows_a, rows_b, acc, tab_spm, sem_a, sem_b):
        cid = lax.axis_index("c")
        sid = lax.axis_index("s")
        wid = sid * 2 + cid
        rpw = NP // 16

        pltpu.sync_copy(zeros_hbm.at[pl.ds(sid * rpw, rpw)],
                        acc.at[pl.ds(sid * rpw, rpw)])

        pltpu.sync_copy(tab_hbm.at[pl.ds(sid * rpw, rpw)],
                        tab_spm.at[pl.ds(sid * rpw, rpw)])
        pltpu.sync_copy(src_hbm.at[wid], src_v)
        pltpu.sync_copy(dst_hbm.at[wid], dst_v)
        plsc.subcore_barrier()

        pltpu.async_copy(tab_spm.at[src_v.at[0]], rows_a, sem_a)

        def body(g, carry):
            j0 = 2 * g
            j1 = j0 + 1
            pltpu.make_async_copy(tab_spm.at[src_v.at[j0]], rows_a, sem_a).wait()
            pltpu.async_copy(tab_spm.at[src_v.at[j1]], rows_b, sem_b)
            pltpu.sync_copy(rows_a, acc.at[dst_v.at[j0]], add=True)
            pltpu.make_async_copy(tab_spm.at[src_v.at[j1]], rows_b, sem_b).wait()
            pltpu.async_copy(tab_spm.at[src_v.at[j0 + 2]], rows_a, sem_a)
            pltpu.sync_copy(rows_b, acc.at[dst_v.at[j1]], add=True)
            return carry

        lax.fori_loop(0, (NCH - 1) // 2, body, 0)
        pltpu.make_async_copy(
            tab_spm.at[src_v.at[NCH - 1]], rows_a, sem_a).wait()
        pltpu.sync_copy(rows_a, acc.at[dst_v.at[NCH - 1]], add=True)
        plsc.subcore_barrier()

        pltpu.sync_copy(acc.at[pl.ds(sid * rpw, rpw)],
                        out_hbm.at[cid].at[pl.ds(sid * rpw, rpw)])

    return agg_kernel


def _mm1_body(x2_ref, w_ref, o_ref):
    o_ref[...] = jnp.dot(x2_ref[...], w_ref[...],
                         preferred_element_type=jnp.float32)


def _scale_body(degp2_ref, h_ref, m2_ref, o_ref):
    deg = degp2_ref[0] + degp2_ref[1] + 1.0
    dp = jnp.dot(lax.rsqrt(deg), m2_ref[...],
                 preferred_element_type=jnp.float32,
                 precision=lax.Precision.HIGHEST)
    o_ref[...] = h_ref[...] * dp


def _mid_body(degp4_ref, s_ref, hs_ref, b_ref, w_ref, m64_ref, m32_ref,
              o_ref):
    deg = degp4_ref[0] + degp4_ref[1] + 1.0
    dinv = lax.rsqrt(deg)
    d64 = jnp.dot(dinv, m64_ref[...], preferred_element_type=jnp.float32,
                  precision=lax.Precision.HIGHEST)
    pre = (s_ref[0] + s_ref[1] + hs_ref[...]) * d64 + b_ref[...]
    hmid = jnp.where(pre >= 0.0, pre, 0.01 * pre)
    h2 = jnp.dot(hmid, w_ref[...], preferred_element_type=jnp.float32)
    d32 = jnp.dot(dinv, m32_ref[...], preferred_element_type=jnp.float32,
                  precision=lax.Precision.HIGHEST)
    o_ref[...] = h2 * d32


def _final_body(degp4_ref, s_ref, hs_ref, b_ref, m32_ref, o_ref):
    deg = degp4_ref[0] + degp4_ref[1] + 1.0
    d32 = jnp.dot(lax.rsqrt(deg), m32_ref[...],
                  preferred_element_type=jnp.float32,
                  precision=lax.Precision.HIGHEST)
    pre = (s_ref[0] + s_ref[1] + hs_ref[...]) * d32 + b_ref[...]
    o_ref[...] = jnp.where(pre >= 0.0, pre, 0.01 * pre)


def kernel(x, edge_index, W1, b1, W2, b2):
    N, d_in = x.shape
    E = edge_index.shape[1]
    h1 = W1.shape[1]
    h2 = W2.shape[1]
    NP = NPAD
    NCH = E // (NW * CW)

    src = edge_index[0].reshape(NW, NCH, CW)
    dst = edge_index[1].reshape(NW, NCH, CW)

    W1d2 = (jnp.zeros((2 * d_in, 2 * h1), jnp.float32)
            .at[:d_in, :h1].set(W1)
            .at[d_in:, h1:].set(W1))
    W2d4 = jnp.zeros((4 * h1, 4 * h2), jnp.float32)
    for j in range(4):
        W2d4 = W2d4.at[j * h1:(j + 1) * h1, j * h2:(j + 1) * h2].set(W2)
    m2 = jnp.repeat(jnp.eye(2, dtype=jnp.float32), h1, axis=1)
    m64 = jnp.repeat(jnp.eye(4, dtype=jnp.float32), h1, axis=1)
    m32 = jnp.repeat(jnp.eye(4, dtype=jnp.float32), h2, axis=1)
    b1q = jnp.tile(b1, 4).reshape(1, 4 * h1)
    b2q = jnp.tile(b2, 4).reshape(1, 4 * h2)

    degp = _make_deg_kernel(NP, NCH)(dst, jnp.zeros((NP,), jnp.float32))
    degp2 = degp.reshape(2, NP // 2, 2)
    degp4 = degp.reshape(2, NP // 4, 4)

    BNp = 1000
    BNq = 640
    x2 = x.reshape(N // 2, 2 * d_in)
    h1p = pl.pallas_call(
        _mm1_body,
        grid=(N // 2 // BNp,),
        in_specs=[
            pl.BlockSpec((BNp, 2 * d_in), lambda i: (i, 0)),
            pl.BlockSpec((2 * d_in, 2 * h1), lambda i: (0, 0)),
        ],
        out_specs=pl.BlockSpec((BNp, 2 * h1), lambda i: (i, 0)),
        out_shape=jax.ShapeDtypeStruct((NP // 2, 2 * h1), jnp.float32),
    )(x2, W1d2)

    hs1p = pl.pallas_call(
        _scale_body,
        grid=(NP // 2 // BNp,),
        in_specs=[
            pl.BlockSpec((2, BNp, 2), lambda i: (0, i, 0)),
            pl.BlockSpec((BNp, 2 * h1), lambda i: (i, 0)),
            pl.BlockSpec((2, 2 * h1), lambda i: (0, 0)),
        ],
        out_specs=pl.BlockSpec((BNp, 2 * h1), lambda i: (i, 0)),
        out_shape=jax.ShapeDtypeStruct((NP // 2, 2 * h1), jnp.float32),
    )(degp2, h1p, m2)

    tab1 = hs1p.reshape(NP, h1)
    s1p = _make_agg_kernel(NP, NCH, h1)(
        tab1, src, dst, jnp.zeros((NP, h1), jnp.float32))

    s1q = s1p.reshape(2, NP // 4, 4 * h1)
    hs1q = hs1p.reshape(NP // 4, 4 * h1)
    hs2q = pl.pallas_call(
        _mid_body,
        grid=(NP // 4 // BNq,),
        in_specs=[
            pl.BlockSpec((2, BNq, 4), lambda i: (0, i, 0)),
            pl.BlockSpec((2, BNq, 4 * h1), lambda i: (0, i, 0)),
            pl.BlockSpec((BNq, 4 * h1), lambda i: (i, 0)),
            pl.BlockSpec((1, 4 * h1), lambda i: (0, 0)),
            pl.BlockSpec((4 * h1, 4 * h2), lambda i: (0, 0)),
            pl.BlockSpec((4, 4 * h1), lambda i: (0, 0)),
            pl.BlockSpec((4, 4 * h2), lambda i: (0, 0)),
        ],
        out_specs=pl.BlockSpec((BNq, 4 * h2), lambda i: (i, 0)),
        out_shape=jax.ShapeDtypeStruct((NP // 4, 4 * h2), jnp.float32),
    )(degp4, s1q, hs1q, b1q, W2d4, m64, m32)

    tab2 = hs2q.reshape(NP, h2)
    s2p = _make_agg_kernel(NP, NCH, h2)(
        tab2, src, dst, jnp.zeros((NP, h2), jnp.float32))

    s2q = s2p.reshape(2, NP // 4, 4 * h2)
    outq = pl.pallas_call(
        _final_body,
        grid=(NP // 4 // BNq,),
        in_specs=[
            pl.BlockSpec((2, BNq, 4), lambda i: (0, i, 0)),
            pl.BlockSpec((2, BNq, 4 * h2), lambda i: (0, i, 0)),
            pl.BlockSpec((BNq, 4 * h2), lambda i: (i, 0)),
            pl.BlockSpec((1, 4 * h2), lambda i: (0, 0)),
            pl.BlockSpec((4, 4 * h2), lambda i: (0, 0)),
        ],
        out_specs=pl.BlockSpec((BNq, 4 * h2), lambda i: (i, 0)),
        out_shape=jax.ShapeDtypeStruct((NP // 4, 4 * h2), jnp.float32),
    )(degp4, s2q, hs2q, b2q, m32)

    return outq[:N // 4].reshape(N, h2)
